# baseline (speedup 1.0000x reference)
import functools

import jax
import jax.numpy as jnp
from jax import lax
from jax.experimental import pallas as pl
from jax.experimental.pallas import tpu as pltpu
from jax.experimental.pallas import tpu_sc as plsc

_L = 16


def _make_sc_mask_kernel(HW, W, bs):
    n_workers = 32
    wslice = HW // n_workers
    pad1 = _L
    pad2 = W * (bs - 1)
    mesh = plsc.VectorSubcoreMesh(core_axis_name="c", subcore_axis_name="s")

    @functools.partial(
        pl.kernel,
        mesh=mesh,
        out_type=jax.ShapeDtypeStruct((HW,), jnp.float32),
        scratch_types=[
            pltpu.VMEM((pad1 + HW,), jnp.float32),
            pltpu.VMEM((pad2 + HW,), jnp.float32),
            pltpu.VMEM((HW,), jnp.float32),
            pltpu.VMEM((3 * _L,), jnp.float32),
        ],
    )
    def sc_mask(seed_hbm, out_hbm, seedp, r1p, bm, tmp):
        wid = lax.axis_index("s") * 2 + lax.axis_index("c")
        zero = jnp.zeros((_L,), jnp.float32)

        def zpad1(i, _):
            seedp[pl.ds(i * _L, _L)] = zero
            return 0

        lax.fori_loop(0, pad1 // _L, zpad1, 0)

        def zpad2(i, _):
            r1p[pl.ds(i * _L, _L)] = zero
            return 0

        lax.fori_loop(0, pad2 // _L, zpad2, 0)

        pltpu.sync_copy(seed_hbm, seedp.at[pl.ds(pad1, HW)])

        def p1(i, _):
            base = i * _L
            wcol = lax.broadcasted_iota(jnp.int32, (_L,), 0) + jnp.full(
                (_L,), base % W, jnp.int32
            )
            r = seedp[pl.ds(pad1 + base, _L)]
            for j in range(1, bs):
                sh = seedp[pl.ds(pad1 + base - j, _L)]
                keep = wcol >= jnp.full((_L,), j, jnp.int32)
                r = jnp.maximum(r, jnp.where(keep, sh, zero))
            r1p[pl.ds(pad2 + base, _L)] = r
            return 0

        lax.fori_loop(0, HW // _L, p1, 0)

        one = jnp.ones((_L,), jnp.float32)

        def p2(i, acc):
            base = i * _L
            b = r1p[pl.ds(pad2 + base, _L)]
            for k in range(1, bs):
                b = jnp.maximum(b, r1p[pl.ds(pad2 + base - W * k, _L)])
            bmv = one - b
            bm[pl.ds(base, _L)] = bmv
            return acc + bmv

        acc = lax.fori_loop(0, HW // _L, p2, jnp.zeros((_L,), jnp.float32))

        tmp[pl.ds(0, _L)] = zero
        tmp[pl.ds(2 * _L, _L)] = zero
        pre = acc
        for sh in (8, 4, 2, 1):
            tmp[pl.ds(_L, _L)] = pre
            pre = tmp[pl.ds(_L, _L)] + tmp[pl.ds(_L - sh, _L)]
        suf = acc
        for sh in (8, 4, 2, 1):
            tmp[pl.ds(_L, _L)] = suf
            suf = tmp[pl.ds(_L, _L)] + tmp[pl.ds(_L + sh, _L)]
        tot = pre + suf - acc
        scale = jnp.full((_L,), float(HW), jnp.float32) / tot

        wbase = wid * wslice

        def scl(i, _):
            off = wbase + i * _L
            bm[pl.ds(off, _L)] = bm[pl.ds(off, _L)] * scale
            return 0

        lax.fori_loop(0, wslice // _L, scl, 0)
        pltpu.sync_copy(bm.at[pl.ds(wbase, wslice)], out_hbm.at[pl.ds(wbase, wslice)])

    return sc_mask


def _tc_mul_body(m_ref, x_ref, o_ref, *, HW, S):
    for k in range(S // HW):
        o_ref[pl.ds(k * HW, HW), :] = x_ref[pl.ds(k * HW, HW), :] * m_ref[:]


def kernel(x, block_size, feat_size, drop_rate):
    B, C, H, W = x.shape
    bs = 7
    gamma = drop_rate / (block_size ** 2) * (
        (feat_size ** 2) / ((feat_size - block_size + 1) ** 2)
    )
    mkey = jax.random.fold_in(jax.random.key(0), 1)
    mask = jax.random.bernoulli(mkey, gamma, (H, W)).astype(jnp.float32)

    HW = H * W
    mcol = _make_sc_mask_kernel(HW, W, bs)(mask.reshape(HW))

    xt = x.transpose(0, 2, 3, 1).reshape(B * HW, C)
    S = HW * 2
    out = pl.pallas_call(
        lambda m_ref, x_ref, o_ref: _tc_mul_body(m_ref, x_ref, o_ref, HW=HW, S=S),
        grid=(B * HW // S,),
        in_specs=[
            pl.BlockSpec((HW, 1), lambda i: (0, 0)),
            pl.BlockSpec((S, C), lambda i: (i, 0)),
        ],
        out_specs=pl.BlockSpec((S, C), lambda i: (i, 0)),
        out_shape=jax.ShapeDtypeStruct((B * HW, C), x.dtype),
        compiler_params=pltpu.CompilerParams(
            dimension_semantics=("arbitrary",),
        ),
    )(mcol.reshape(HW, 1), xt)
    return out.reshape(B, H, W, C).transpose(0, 3, 1, 2)


# speedup vs baseline: 1.0556x; 1.0556x over previous
import functools

import jax
import jax.numpy as jnp
from jax import lax
from jax.experimental import pallas as pl
from jax.experimental.pallas import tpu as pltpu
from jax.experimental.pallas import tpu_sc as plsc

_L = 16


def _make_sc_mask_kernel(HW, W, bs):
    n_sub = 16
    tslice = HW // n_sub
    pad1 = W * (bs - 1) + _L
    pad2 = W * (bs - 1)
    halo = W * (bs - 1)
    mesh = plsc.VectorSubcoreMesh(core_axis_name="c", subcore_axis_name="s")

    @functools.partial(
        pl.kernel,
        mesh=mesh,
        out_type=jax.ShapeDtypeStruct((HW,), jnp.float32),
        scratch_types=[
            pltpu.VMEM((pad1 + HW,), jnp.float32),
            pltpu.VMEM((pad2 + tslice,), jnp.float32),
            pltpu.VMEM((tslice,), jnp.float32),
            pltpu.VMEM((3 * _L,), jnp.float32),
            pltpu.VMEM((n_sub * _L,), jnp.float32),
            pltpu.VMEM_SHARED((n_sub * _L,), jnp.float32),
        ],
    )
    def sc_mask(seed_hbm, out_hbm, seedp, r1p, bm, tmp, psum, shared):
        cid = lax.axis_index("c")
        sid = lax.axis_index("s")
        sbase = sid * tslice
        zero = jnp.zeros((_L,), jnp.float32)

        def zpad(i, _):
            seedp[pl.ds(i * _L, _L)] = zero
            return 0

        lax.fori_loop(0, pad1 // _L, zpad, 0)

        pltpu.sync_copy(seed_hbm, seedp.at[pl.ds(pad1, HW)])

        def p1(i, _):
            base = i * _L
            wcol = lax.broadcasted_iota(jnp.int32, (_L,), 0) + jnp.full(
                (_L,), base % W, jnp.int32
            )
            src = pad1 + sbase - halo + base
            r = seedp[pl.ds(src, _L)]
            for j in range(1, bs):
                sh = seedp[pl.ds(src - j, _L)]
                keep = wcol >= jnp.full((_L,), j, jnp.int32)
                r = jnp.maximum(r, jnp.where(keep, sh, zero))
            r1p[pl.ds(base, _L)] = r
            return 0

        lax.fori_loop(0, (halo + tslice) // _L, p1, 0)

        one = jnp.ones((_L,), jnp.float32)

        def p2(i, acc):
            base = i * _L
            b = r1p[pl.ds(pad2 + base, _L)]
            for k in range(1, bs):
                b = jnp.maximum(b, r1p[pl.ds(pad2 + base - W * k, _L)])
            bmv = one - b
            bm[pl.ds(base, _L)] = bmv
            return acc + bmv

        acc = lax.fori_loop(0, tslice // _L, p2, jnp.zeros((_L,), jnp.float32))

        psum[pl.ds(0, _L)] = acc
        pltpu.sync_copy(psum.at[pl.ds(0, _L)], shared.at[pl.ds(sid * _L, _L)])
        plsc.subcore_barrier()
        pltpu.sync_copy(shared, psum)

        def psum_red(i, a):
            return a + psum[pl.ds(i * _L, _L)]

        acc = lax.fori_loop(0, n_sub, psum_red, jnp.zeros((_L,), jnp.float32))

        tmp[pl.ds(0, _L)] = zero
        tmp[pl.ds(2 * _L, _L)] = zero
        pre = acc
        for sh in (8, 4, 2, 1):
            tmp[pl.ds(_L, _L)] = pre
            pre = tmp[pl.ds(_L, _L)] + tmp[pl.ds(_L - sh, _L)]
        suf = acc
        for sh in (8, 4, 2, 1):
            tmp[pl.ds(_L, _L)] = suf
            suf = tmp[pl.ds(_L, _L)] + tmp[pl.ds(_L + sh, _L)]
        tot = pre + suf - acc
        scale = jnp.full((_L,), float(HW), jnp.float32) / tot

        half = tslice // 2
        woff = cid * half

        def scl(i, _):
            off = woff + i * _L
            bm[pl.ds(off, _L)] = bm[pl.ds(off, _L)] * scale
            return 0

        lax.fori_loop(0, half // _L, scl, 0)
        pltpu.sync_copy(
            bm.at[pl.ds(woff, half)], out_hbm.at[pl.ds(sbase + woff, half)]
        )

    return sc_mask


def _tc_mul_body(m_ref, x_ref, o_ref, *, HW, S):
    for k in range(S // HW):
        o_ref[pl.ds(k * HW, HW), :] = x_ref[pl.ds(k * HW, HW), :] * m_ref[:]


def kernel(x, block_size, feat_size, drop_rate):
    B, C, H, W = x.shape
    bs = 7
    gamma = drop_rate / (block_size ** 2) * (
        (feat_size ** 2) / ((feat_size - block_size + 1) ** 2)
    )
    mkey = jax.random.fold_in(jax.random.key(0), 1)
    mask = jax.random.bernoulli(mkey, gamma, (H, W)).astype(jnp.float32)

    HW = H * W
    mcol = _make_sc_mask_kernel(HW, W, bs)(mask.reshape(HW))

    xt = x.transpose(0, 2, 3, 1).reshape(B * HW, C)
    S = HW * 2
    out = pl.pallas_call(
        lambda m_ref, x_ref, o_ref: _tc_mul_body(m_ref, x_ref, o_ref, HW=HW, S=S),
        grid=(B * HW // S,),
        in_specs=[
            pl.BlockSpec((HW, 1), lambda i: (0, 0)),
            pl.BlockSpec((S, C), lambda i: (i, 0)),
        ],
        out_specs=pl.BlockSpec((S, C), lambda i: (i, 0)),
        out_shape=jax.ShapeDtypeStruct((B * HW, C), x.dtype),
        compiler_params=pltpu.CompilerParams(
            dimension_semantics=("arbitrary",),
        ),
    )(mcol.reshape(HW, 1), xt)
    return out.reshape(B, H, W, C).transpose(0, 3, 1, 2)
